# baseline (speedup 1.0000x reference)
import functools

import jax
import jax.numpy as jnp
from jax import lax
from jax.experimental import pallas as pl
from jax.experimental.pallas import tpu as pltpu
from jax.experimental.pallas import tpu_sc as plsc

N = 10000
NP = 10240
E = 320000
D = 128

NC = 2
NS = 16
NW = NC * NS
K = 125
CH = E // (NW * K)
HC = CH // 2
RS = NP // NS

_MESH = plsc.VectorSubcoreMesh(
    core_axis_name="c", subcore_axis_name="s", num_cores=NC, num_subcores=NS
)


@functools.partial(
    pl.kernel,
    out_type=jax.ShapeDtypeStruct((NC, NP, D), jnp.float32),
    mesh=_MESH,
    scratch_types=[
        pltpu.VMEM((CH, K), jnp.int32),
        pltpu.VMEM((K, D), jnp.float32),
        pltpu.VMEM_SHARED((NP, D), jnp.float32),
    ],
)
def _sc_count(dst_hbm, ones_hbm, z128_hbm, out_hbm, dst_v, ones_v, acc_sh):
    c = lax.axis_index("c")
    s = lax.axis_index("s")
    wid = c * NS + s
    pltpu.sync_copy(z128_hbm, acc_sh.at[pl.ds(s * RS, RS)])
    pltpu.sync_copy(dst_hbm.at[wid], dst_v)
    pltpu.sync_copy(ones_hbm, ones_v)
    plsc.subcore_barrier()

    def body(j, carry):
        pltpu.sync_copy(ones_v, acc_sh.at[dst_v.at[j]], add=True)
        return carry

    lax.fori_loop(0, CH, body, 0)
    plsc.subcore_barrier()
    pltpu.sync_copy(acc_sh.at[pl.ds(s * RS, RS)], out_hbm.at[c, pl.ds(s * RS, RS)])


@functools.partial(
    pl.kernel,
    out_type=jax.ShapeDtypeStruct((NC, NP, D), jnp.float32),
    mesh=_MESH,
    scratch_types=[
        pltpu.VMEM((2 * HC, K), jnp.int32),
        pltpu.VMEM((2, K, D), jnp.float32),
        pltpu.VMEM_SHARED((NP, D), jnp.float32),
        pltpu.SemaphoreType.DMA,
        pltpu.SemaphoreType.DMA,
        pltpu.SemaphoreType.DMA,
        pltpu.SemaphoreType.DMA,
    ],
)
def _sc_agg(h_hbm, src_hbm, dst_hbm, z128_hbm, out_hbm,
            idx_v, rows, acc_sh, semg0, semg1, sems0, sems1):
    c = lax.axis_index("c")
    s = lax.axis_index("s")
    wid = c * NS + s
    pltpu.sync_copy(z128_hbm, acc_sh.at[pl.ds(s * RS, RS)])
    plsc.subcore_barrier()

    for half in range(2):
        base = half * HC
        pltpu.sync_copy(src_hbm.at[wid, pl.ds(base, HC)], idx_v.at[pl.ds(0, HC)])
        pltpu.sync_copy(dst_hbm.at[wid, pl.ds(base, HC)], idx_v.at[pl.ds(HC, HC)])
        pltpu.async_copy(h_hbm.at[idx_v.at[0]], rows.at[0], semg0)
        pltpu.async_copy(h_hbm.at[idx_v.at[1]], rows.at[1], semg1)

        def body(jj, carry):
            l0 = jj * 2
            pltpu.make_async_copy(h_hbm.at[idx_v.at[l0]], rows.at[0], semg0).wait()
            pltpu.async_copy(rows.at[0], acc_sh.at[idx_v.at[HC + l0]], sems0,
                             add=True)
            pltpu.make_async_copy(h_hbm.at[idx_v.at[l0 + 1]], rows.at[1],
                                  semg1).wait()
            pltpu.async_copy(rows.at[1], acc_sh.at[idx_v.at[HC + l0 + 1]], sems1,
                             add=True)
            pltpu.make_async_copy(rows.at[0], acc_sh.at[idx_v.at[HC + l0]],
                                  sems0).wait()

            @pl.when(l0 + 2 < HC)
            def _():
                pltpu.async_copy(h_hbm.at[idx_v.at[l0 + 2]], rows.at[0], semg0)

            pltpu.make_async_copy(rows.at[1], acc_sh.at[idx_v.at[HC + l0 + 1]],
                                  sems1).wait()

            @pl.when(l0 + 3 < HC)
            def _():
                pltpu.async_copy(h_hbm.at[idx_v.at[l0 + 3]], rows.at[1], semg1)

            return carry

        lax.fori_loop(0, HC // 2, body, 0)
    plsc.subcore_barrier()
    pltpu.sync_copy(acc_sh.at[pl.ds(s * RS, RS)], out_hbm.at[c, pl.ds(s * RS, RS)])


_B = 1024


def _mm1_body(cnt_ref, x_ref, w_ref, hp_ref, dinv_ref):
    p = cnt_ref[...]
    deg = 1.0 + p[0, :, 0:1] + p[1, :, 0:1]
    dv = lax.rsqrt(deg)
    h = jnp.dot(x_ref[...], w_ref[...], preferred_element_type=jnp.float32)
    hp_ref[...] = dv * h
    dinv_ref[...] = dv


def _mm2_body(p_ref, hp_ref, dinv_ref, b1_ref, w_ref, out_ref):
    p = p_ref[...]
    dv = dinv_ref[...]
    t = p[0] + p[1] + hp_ref[...]
    z = jnp.maximum(dv * t + b1_ref[...], 0.0)
    out_ref[...] = dv * jnp.dot(z, w_ref[...], preferred_element_type=jnp.float32)


def _mm3_body(p_ref, hp_ref, dinv_ref, b2_ref, w_ref, bm_ref, out_ref):
    p = p_ref[...]
    t = p[0] + p[1] + hp_ref[...]
    h2 = dinv_ref[...] * t + b2_ref[...]
    out_ref[...] = (
        jnp.dot(h2, w_ref[...], preferred_element_type=jnp.float32) + bm_ref[...]
    )


def _row_block(shape_last):
    return pl.BlockSpec((_B, shape_last), lambda i: (i, 0))


_P128_SPEC = pl.BlockSpec((NC, _B, D), lambda i: (0, i, 0))
_W_SPEC = pl.BlockSpec((D, D), lambda i: (0, 0))
_BIAS_SPEC = pl.BlockSpec((1, D), lambda i: (0, 0))
_GRID = (NP // _B,)

_mm1 = pl.pallas_call(
    _mm1_body,
    grid=_GRID,
    in_specs=[_P128_SPEC, _row_block(D), _W_SPEC],
    out_specs=[_row_block(D), _row_block(1)],
    out_shape=[
        jax.ShapeDtypeStruct((NP, D), jnp.float32),
        jax.ShapeDtypeStruct((NP, 1), jnp.float32),
    ],
)

_mm2 = pl.pallas_call(
    _mm2_body,
    grid=_GRID,
    in_specs=[_P128_SPEC, _row_block(D), _row_block(1), _BIAS_SPEC, _W_SPEC],
    out_specs=_row_block(D),
    out_shape=jax.ShapeDtypeStruct((NP, D), jnp.float32),
)

_mm3 = pl.pallas_call(
    _mm3_body,
    grid=_GRID,
    in_specs=[_P128_SPEC, _row_block(D), _row_block(1), _BIAS_SPEC, _W_SPEC,
              _BIAS_SPEC],
    out_specs=_row_block(D),
    out_shape=jax.ShapeDtypeStruct((NP, D), jnp.float32),
)


def kernel(x, edge_index, W1, b1, W2, b2, Wm, bm):
    ei = edge_index.astype(jnp.int32)
    src3 = ei[0].reshape(NW, CH, K)
    dst3 = ei[1].reshape(NW, CH, K)
    ones128 = jnp.ones((K, D), jnp.float32)
    z128 = jnp.zeros((RS, D), jnp.float32)

    xp = jnp.pad(x, ((0, NP - N), (0, 0)))
    cnt = _sc_count(dst3, ones128, z128)
    h1p, dinv = _mm1(cnt, xp, W1)
    p1 = _sc_agg(h1p, src3, dst3, z128)
    h2p = _mm2(p1, h1p, dinv, b1.reshape(1, D), W2)
    p2 = _sc_agg(h2p, src3, dst3, z128)
    out = _mm3(p2, h2p, dinv, b2.reshape(1, D), Wm, bm.reshape(1, D))
    return out[:N]


# speedup vs baseline: 1.1907x; 1.1907x over previous
import functools

import jax
import jax.numpy as jnp
from jax import lax
from jax.experimental import pallas as pl
from jax.experimental.pallas import tpu as pltpu
from jax.experimental.pallas import tpu_sc as plsc

N = 10000
NP = 10240
E = 320000
D = 128

NC = 2
NS = 16
NW = NC * NS
K = 125
CH = E // (NW * K)
HC = CH // 2
RS = NP // NS

_MESH = plsc.VectorSubcoreMesh(
    core_axis_name="c", subcore_axis_name="s", num_cores=NC, num_subcores=NS
)


@functools.partial(
    pl.kernel,
    out_type=jax.ShapeDtypeStruct((NC, NP, D), jnp.float32),
    mesh=_MESH,
    scratch_types=[
        pltpu.VMEM((CH, K), jnp.int32),
        pltpu.VMEM((K, D), jnp.float32),
        pltpu.VMEM_SHARED((NP, D), jnp.float32),
    ],
)
def _sc_count(dst_hbm, ones_hbm, z128_hbm, out_hbm, dst_v, ones_v, acc_sh):
    c = lax.axis_index("c")
    s = lax.axis_index("s")
    wid = c * NS + s
    pltpu.sync_copy(z128_hbm, acc_sh.at[pl.ds(s * RS, RS)])
    pltpu.sync_copy(dst_hbm.at[wid], dst_v)
    pltpu.sync_copy(ones_hbm, ones_v)
    plsc.subcore_barrier()

    def body(j, carry):
        pltpu.sync_copy(ones_v, acc_sh.at[dst_v.at[j]], add=True)
        return carry

    lax.fori_loop(0, CH, body, 0)
    plsc.subcore_barrier()
    pltpu.sync_copy(acc_sh.at[pl.ds(s * RS, RS)], out_hbm.at[c, pl.ds(s * RS, RS)])


@functools.partial(
    pl.kernel,
    out_type=jax.ShapeDtypeStruct((NC, NP, D), jnp.float32),
    mesh=_MESH,
    scratch_types=[
        pltpu.VMEM((2 * HC, K), jnp.int32),
        pltpu.VMEM((2, K, D), jnp.float32),
        pltpu.VMEM_SHARED((NP, D), jnp.float32),
        pltpu.SemaphoreType.DMA,
        pltpu.SemaphoreType.DMA,
    ],
)
def _sc_agg(h_hbm, src_hbm, dst_hbm, z128_hbm, out_hbm,
            idx_v, rows, acc_sh, sem0, sem1):
    c = lax.axis_index("c")
    s = lax.axis_index("s")
    wid = c * NS + s
    pltpu.sync_copy(z128_hbm, acc_sh.at[pl.ds(s * RS, RS)])
    plsc.subcore_barrier()

    for half in range(2):
        base = half * HC
        pltpu.sync_copy(src_hbm.at[wid, pl.ds(base, HC)], idx_v.at[pl.ds(0, HC)])
        pltpu.sync_copy(dst_hbm.at[wid, pl.ds(base, HC)], idx_v.at[pl.ds(HC, HC)])
        pltpu.async_copy(h_hbm.at[idx_v.at[0]], rows.at[0], sem0)

        def body(jj, carry):
            l0 = jj * 2
            pltpu.async_copy(h_hbm.at[idx_v.at[l0 + 1]], rows.at[1], sem1)
            pltpu.make_async_copy(h_hbm.at[idx_v.at[l0]], rows.at[0], sem0).wait()
            pltpu.sync_copy(rows.at[0], acc_sh.at[idx_v.at[HC + l0]], add=True)

            @pl.when(jj + 1 < HC // 2)
            def _():
                pltpu.async_copy(h_hbm.at[idx_v.at[l0 + 2]], rows.at[0], sem0)

            pltpu.make_async_copy(h_hbm.at[idx_v.at[l0 + 1]], rows.at[1], sem1).wait()
            pltpu.sync_copy(rows.at[1], acc_sh.at[idx_v.at[HC + l0 + 1]], add=True)
            return carry

        lax.fori_loop(0, HC // 2, body, 0)
    plsc.subcore_barrier()
    pltpu.sync_copy(acc_sh.at[pl.ds(s * RS, RS)], out_hbm.at[c, pl.ds(s * RS, RS)])


_B = 1024


def _mm1_body(cnt_ref, x_ref, w_ref, hp_ref, dinv_ref):
    p = cnt_ref[...]
    deg = 1.0 + p[0, :, 0:1] + p[1, :, 0:1]
    dv = lax.rsqrt(deg)
    h = jnp.dot(x_ref[...], w_ref[...], preferred_element_type=jnp.float32)
    hp_ref[...] = dv * h
    dinv_ref[...] = dv


def _mm2_body(p_ref, hp_ref, dinv_ref, b1_ref, w_ref, out_ref):
    p = p_ref[...]
    dv = dinv_ref[...]
    t = p[0] + p[1] + hp_ref[...]
    z = jnp.maximum(dv * t + b1_ref[...], 0.0)
    out_ref[...] = dv * jnp.dot(z, w_ref[...], preferred_element_type=jnp.float32)


def _mm3_body(p_ref, hp_ref, dinv_ref, b2_ref, w_ref, bm_ref, out_ref):
    p = p_ref[...]
    t = p[0] + p[1] + hp_ref[...]
    h2 = dinv_ref[...] * t + b2_ref[...]
    out_ref[...] = (
        jnp.dot(h2, w_ref[...], preferred_element_type=jnp.float32) + bm_ref[...]
    )


def _row_block(shape_last):
    return pl.BlockSpec((_B, shape_last), lambda i: (i, 0))


_P128_SPEC = pl.BlockSpec((NC, _B, D), lambda i: (0, i, 0))
_W_SPEC = pl.BlockSpec((D, D), lambda i: (0, 0))
_BIAS_SPEC = pl.BlockSpec((1, D), lambda i: (0, 0))
_GRID = (NP // _B,)

_mm1 = pl.pallas_call(
    _mm1_body,
    grid=_GRID,
    in_specs=[_P128_SPEC, _row_block(D), _W_SPEC],
    out_specs=[_row_block(D), _row_block(1)],
    out_shape=[
        jax.ShapeDtypeStruct((NP, D), jnp.float32),
        jax.ShapeDtypeStruct((NP, 1), jnp.float32),
    ],
)

_mm2 = pl.pallas_call(
    _mm2_body,
    grid=_GRID,
    in_specs=[_P128_SPEC, _row_block(D), _row_block(1), _BIAS_SPEC, _W_SPEC],
    out_specs=_row_block(D),
    out_shape=jax.ShapeDtypeStruct((NP, D), jnp.float32),
)

_mm3 = pl.pallas_call(
    _mm3_body,
    grid=_GRID,
    in_specs=[_P128_SPEC, _row_block(D), _row_block(1), _BIAS_SPEC, _W_SPEC,
              _BIAS_SPEC],
    out_specs=_row_block(D),
    out_shape=jax.ShapeDtypeStruct((NP, D), jnp.float32),
)


def kernel(x, edge_index, W1, b1, W2, b2, Wm, bm):
    ei = edge_index.astype(jnp.int32)
    src3 = ei[0].reshape(NW, CH, K)
    dst3 = ei[1].reshape(NW, CH, K)
    ones128 = jnp.ones((K, D), jnp.float32)
    z128 = jnp.zeros((RS, D), jnp.float32)

    xp = jnp.pad(x, ((0, NP - N), (0, 0)))
    cnt = _sc_count(dst3, ones128, z128)
    h1p, dinv = _mm1(cnt, xp, W1)
    p1 = _sc_agg(h1p, src3, dst3, z128)
    h2p = _mm2(p1, h1p, dinv, b1.reshape(1, D), W2)
    p2 = _sc_agg(h2p, src3, dst3, z128)
    out = _mm3(p2, h2p, dinv, b2.reshape(1, D), Wm, bm.reshape(1, D))
    return out[:N]


# speedup vs baseline: 1.1998x; 1.0077x over previous
import functools

import jax
import jax.numpy as jnp
from jax import lax
from jax.experimental import pallas as pl
from jax.experimental.pallas import tpu as pltpu
from jax.experimental.pallas import tpu_sc as plsc

N = 10000
NP = 10240
E = 320000
D = 128

NC = 2
NS = 16
NW = NC * NS
K = 125
CH = E // (NW * K)
HC = CH // 2
RS = NP // NS

_MESH = plsc.VectorSubcoreMesh(
    core_axis_name="c", subcore_axis_name="s", num_cores=NC, num_subcores=NS
)


@functools.partial(
    pl.kernel,
    out_type=jax.ShapeDtypeStruct((NC, NP, D), jnp.float32),
    mesh=_MESH,
    scratch_types=[
        pltpu.VMEM((CH, K), jnp.int32),
        pltpu.VMEM((K, D), jnp.float32),
        pltpu.VMEM_SHARED((NP, D), jnp.float32),
        pltpu.SemaphoreType.DMA,
        pltpu.SemaphoreType.DMA,
    ],
)
def _sc_count(dst_hbm, ones_hbm, z128_hbm, out_hbm, dst_v, ones_v, acc_sh,
              sem, semp):
    c = lax.axis_index("c")
    s = lax.axis_index("s")
    wid = c * NS + s
    za = pltpu.async_copy(z128_hbm, acc_sh.at[pl.ds(s * RS, RS)], semp)
    pltpu.sync_copy(dst_hbm.at[wid], dst_v)
    pltpu.sync_copy(ones_hbm, ones_v)
    za.wait()
    plsc.subcore_barrier()

    def body(j, carry):
        pltpu.async_copy(ones_v, acc_sh.at[dst_v.at[j]], sem, add=True)

        @pl.when(j >= 2)
        def _():
            pltpu.make_async_copy(ones_v, acc_sh.at[dst_v.at[j - 2]], sem).wait()

        return carry

    lax.fori_loop(0, CH, body, 0)
    pltpu.make_async_copy(ones_v, acc_sh.at[dst_v.at[CH - 2]], sem).wait()
    pltpu.make_async_copy(ones_v, acc_sh.at[dst_v.at[CH - 1]], sem).wait()
    plsc.subcore_barrier()
    pltpu.sync_copy(acc_sh.at[pl.ds(s * RS, RS)], out_hbm.at[c, pl.ds(s * RS, RS)])


@functools.partial(
    pl.kernel,
    out_type=jax.ShapeDtypeStruct((NC, NP, D), jnp.float32),
    mesh=_MESH,
    scratch_types=[
        pltpu.VMEM((2 * HC, K), jnp.int32),
        pltpu.VMEM((2, K, D), jnp.float32),
        pltpu.VMEM_SHARED((NP, D), jnp.float32),
        pltpu.SemaphoreType.DMA,
        pltpu.SemaphoreType.DMA,
    ],
)
def _sc_agg(h_hbm, src_hbm, dst_hbm, z128_hbm, out_hbm,
            idx_v, rows, acc_sh, sem0, sem1):
    c = lax.axis_index("c")
    s = lax.axis_index("s")
    wid = c * NS + s
    pltpu.sync_copy(z128_hbm, acc_sh.at[pl.ds(s * RS, RS)])
    plsc.subcore_barrier()

    for half in range(2):
        base = half * HC
        pltpu.sync_copy(src_hbm.at[wid, pl.ds(base, HC)], idx_v.at[pl.ds(0, HC)])
        pltpu.sync_copy(dst_hbm.at[wid, pl.ds(base, HC)], idx_v.at[pl.ds(HC, HC)])
        pltpu.async_copy(h_hbm.at[idx_v.at[0]], rows.at[0], sem0)

        def body(jj, carry):
            l0 = jj * 2
            pltpu.async_copy(h_hbm.at[idx_v.at[l0 + 1]], rows.at[1], sem1)
            pltpu.make_async_copy(h_hbm.at[idx_v.at[l0]], rows.at[0], sem0).wait()
            pltpu.sync_copy(rows.at[0], acc_sh.at[idx_v.at[HC + l0]], add=True)

            @pl.when(jj + 1 < HC // 2)
            def _():
                pltpu.async_copy(h_hbm.at[idx_v.at[l0 + 2]], rows.at[0], sem0)

            pltpu.make_async_copy(h_hbm.at[idx_v.at[l0 + 1]], rows.at[1], sem1).wait()
            pltpu.sync_copy(rows.at[1], acc_sh.at[idx_v.at[HC + l0 + 1]], add=True)
            return carry

        lax.fori_loop(0, HC // 2, body, 0)
    plsc.subcore_barrier()
    pltpu.sync_copy(acc_sh.at[pl.ds(s * RS, RS)], out_hbm.at[c, pl.ds(s * RS, RS)])


_B = 1024


def _mm1_body(cnt_ref, x_ref, w_ref, hp_ref, dinv_ref):
    p = cnt_ref[...]
    deg = 1.0 + p[0, :, 0:1] + p[1, :, 0:1]
    dv = lax.rsqrt(deg)
    h = jnp.dot(x_ref[...], w_ref[...], preferred_element_type=jnp.float32)
    hp_ref[...] = dv * h
    dinv_ref[...] = dv


def _mm2_body(p_ref, hp_ref, dinv_ref, b1_ref, w_ref, out_ref):
    p = p_ref[...]
    dv = dinv_ref[...]
    t = p[0] + p[1] + hp_ref[...]
    z = jnp.maximum(dv * t + b1_ref[...], 0.0)
    out_ref[...] = dv * jnp.dot(z, w_ref[...], preferred_element_type=jnp.float32)


def _mm3_body(p_ref, hp_ref, dinv_ref, b2_ref, w_ref, bm_ref, out_ref):
    p = p_ref[...]
    t = p[0] + p[1] + hp_ref[...]
    h2 = dinv_ref[...] * t + b2_ref[...]
    out_ref[...] = (
        jnp.dot(h2, w_ref[...], preferred_element_type=jnp.float32) + bm_ref[...]
    )


def _row_block(shape_last):
    return pl.BlockSpec((_B, shape_last), lambda i: (i, 0))


_P128_SPEC = pl.BlockSpec((NC, _B, D), lambda i: (0, i, 0))
_W_SPEC = pl.BlockSpec((D, D), lambda i: (0, 0))
_BIAS_SPEC = pl.BlockSpec((1, D), lambda i: (0, 0))
_GRID = (NP // _B,)

_mm1 = pl.pallas_call(
    _mm1_body,
    grid=_GRID,
    in_specs=[_P128_SPEC, _row_block(D), _W_SPEC],
    out_specs=[_row_block(D), _row_block(1)],
    out_shape=[
        jax.ShapeDtypeStruct((NP, D), jnp.float32),
        jax.ShapeDtypeStruct((NP, 1), jnp.float32),
    ],
)

_mm2 = pl.pallas_call(
    _mm2_body,
    grid=_GRID,
    in_specs=[_P128_SPEC, _row_block(D), _row_block(1), _BIAS_SPEC, _W_SPEC],
    out_specs=_row_block(D),
    out_shape=jax.ShapeDtypeStruct((NP, D), jnp.float32),
)

_mm3 = pl.pallas_call(
    _mm3_body,
    grid=_GRID,
    in_specs=[_P128_SPEC, _row_block(D), _row_block(1), _BIAS_SPEC, _W_SPEC,
              _BIAS_SPEC],
    out_specs=_row_block(D),
    out_shape=jax.ShapeDtypeStruct((NP, D), jnp.float32),
)


def kernel(x, edge_index, W1, b1, W2, b2, Wm, bm):
    ei = edge_index.astype(jnp.int32)
    src3 = ei[0].reshape(NW, CH, K)
    dst3 = ei[1].reshape(NW, CH, K)
    ones128 = jnp.ones((K, D), jnp.float32)
    z128 = jnp.zeros((RS, D), jnp.float32)

    xp = jnp.pad(x, ((0, NP - N), (0, 0)))
    cnt = _sc_count(dst3, ones128, z128)
    h1p, dinv = _mm1(cnt, xp, W1)
    p1 = _sc_agg(h1p, src3, dst3, z128)
    h2p = _mm2(p1, h1p, dinv, b1.reshape(1, D), W2)
    p2 = _sc_agg(h2p, src3, dst3, z128)
    out = _mm3(p2, h2p, dinv, b2.reshape(1, D), Wm, bm.reshape(1, D))
    return out[:N]


# speedup vs baseline: 1.2160x; 1.0134x over previous
import functools

import jax
import jax.numpy as jnp
from jax import lax
from jax.experimental import pallas as pl
from jax.experimental.pallas import tpu as pltpu
from jax.experimental.pallas import tpu_sc as plsc

N = 10000
NP = 10240
E = 320000
D = 128

NC = 2
NS = 16
NW = NC * NS
K = 125
CH = E // (NW * K)
HC = CH // 2
RS = NP // NS

_MESH = plsc.VectorSubcoreMesh(
    core_axis_name="c", subcore_axis_name="s", num_cores=NC, num_subcores=NS
)


@functools.partial(
    pl.kernel,
    out_type=jax.ShapeDtypeStruct((NC, NP, D), jnp.float32),
    mesh=_MESH,
    scratch_types=[
        pltpu.VMEM((CH, K), jnp.int32),
        pltpu.VMEM((K, D), jnp.float32),
        pltpu.VMEM_SHARED((NP, D), jnp.float32),
        pltpu.SemaphoreType.DMA,
        pltpu.SemaphoreType.DMA,
    ],
)
def _sc_count(dst_hbm, ones_hbm, z128_hbm, out_hbm, dst_v, ones_v, acc_sh,
              sem, semp):
    c = lax.axis_index("c")
    s = lax.axis_index("s")
    wid = c * NS + s
    za = pltpu.async_copy(z128_hbm, acc_sh.at[pl.ds(s * RS, RS)], semp)
    pltpu.sync_copy(dst_hbm.at[wid], dst_v)
    pltpu.sync_copy(ones_hbm, ones_v)
    za.wait()
    plsc.subcore_barrier()

    def body(j, carry):
        pltpu.async_copy(ones_v, acc_sh.at[dst_v.at[j]], sem, add=True)

        @pl.when(j >= 2)
        def _():
            pltpu.make_async_copy(ones_v, acc_sh.at[dst_v.at[j - 2]], sem).wait()

        return carry

    lax.fori_loop(0, CH, body, 0)
    pltpu.make_async_copy(ones_v, acc_sh.at[dst_v.at[CH - 2]], sem).wait()
    pltpu.make_async_copy(ones_v, acc_sh.at[dst_v.at[CH - 1]], sem).wait()
    plsc.subcore_barrier()
    pltpu.sync_copy(acc_sh.at[pl.ds(s * RS, RS)], out_hbm.at[c, pl.ds(s * RS, RS)])


@functools.partial(
    pl.kernel,
    out_type=jax.ShapeDtypeStruct((NC, NP, D), jnp.float32),
    mesh=_MESH,
    scratch_types=[
        pltpu.VMEM((2 * HC, K), jnp.int32),
        pltpu.VMEM((2, K, D), jnp.float32),
        pltpu.VMEM_SHARED((NP, D), jnp.float32),
        pltpu.SemaphoreType.DMA,
        pltpu.SemaphoreType.DMA,
        pltpu.SemaphoreType.DMA,
    ],
)
def _sc_agg(h_hbm, src_hbm, dst_hbm, z128_hbm, out_hbm,
            idx_v, rows, acc_sh, sem0, sem1, semp):
    c = lax.axis_index("c")
    s = lax.axis_index("s")
    wid = c * NS + s
    za = pltpu.async_copy(z128_hbm, acc_sh.at[pl.ds(s * RS, RS)], semp)

    for half in range(2):
        base = half * HC
        pltpu.sync_copy(src_hbm.at[wid, pl.ds(base, HC)], idx_v.at[pl.ds(0, HC)])
        pltpu.sync_copy(dst_hbm.at[wid, pl.ds(base, HC)], idx_v.at[pl.ds(HC, HC)])
        pltpu.async_copy(h_hbm.at[idx_v.at[0]], rows.at[0], sem0)
        if half == 0:
            za.wait()
            plsc.subcore_barrier()

        def body(jj, carry):
            l0 = jj * 2
            pltpu.async_copy(h_hbm.at[idx_v.at[l0 + 1]], rows.at[1], sem1)
            pltpu.make_async_copy(h_hbm.at[idx_v.at[l0]], rows.at[0], sem0).wait()
            pltpu.sync_copy(rows.at[0], acc_sh.at[idx_v.at[HC + l0]], add=True)

            @pl.when(jj + 1 < HC // 2)
            def _():
                pltpu.async_copy(h_hbm.at[idx_v.at[l0 + 2]], rows.at[0], sem0)

            pltpu.make_async_copy(h_hbm.at[idx_v.at[l0 + 1]], rows.at[1], sem1).wait()
            pltpu.sync_copy(rows.at[1], acc_sh.at[idx_v.at[HC + l0 + 1]], add=True)
            return carry

        lax.fori_loop(0, HC // 2, body, 0)
    plsc.subcore_barrier()
    pltpu.sync_copy(acc_sh.at[pl.ds(s * RS, RS)], out_hbm.at[c, pl.ds(s * RS, RS)])


_B = 1024


def _mm1_body(cnt_ref, x_ref, w_ref, hp_ref, dinv_ref):
    p = cnt_ref[...]
    deg = 1.0 + p[0, :, 0:1] + p[1, :, 0:1]
    dv = lax.rsqrt(deg)
    h = jnp.dot(x_ref[...], w_ref[...], preferred_element_type=jnp.float32)
    hp_ref[...] = dv * h
    dinv_ref[...] = dv


def _mm2_body(p_ref, hp_ref, dinv_ref, b1_ref, w_ref, out_ref):
    p = p_ref[...]
    dv = dinv_ref[...]
    t = p[0] + p[1] + hp_ref[...]
    z = jnp.maximum(dv * t + b1_ref[...], 0.0)
    out_ref[...] = dv * jnp.dot(z, w_ref[...], preferred_element_type=jnp.float32)


def _mm3_body(p_ref, hp_ref, dinv_ref, b2_ref, w_ref, bm_ref, out_ref):
    p = p_ref[...]
    t = p[0] + p[1] + hp_ref[...]
    h2 = dinv_ref[...] * t + b2_ref[...]
    out_ref[...] = (
        jnp.dot(h2, w_ref[...], preferred_element_type=jnp.float32) + bm_ref[...]
    )


def _row_block(shape_last):
    return pl.BlockSpec((_B, shape_last), lambda i: (i, 0))


_P128_SPEC = pl.BlockSpec((NC, _B, D), lambda i: (0, i, 0))
_W_SPEC = pl.BlockSpec((D, D), lambda i: (0, 0))
_BIAS_SPEC = pl.BlockSpec((1, D), lambda i: (0, 0))
_GRID = (NP // _B,)

_mm1 = pl.pallas_call(
    _mm1_body,
    grid=_GRID,
    in_specs=[_P128_SPEC, _row_block(D), _W_SPEC],
    out_specs=[_row_block(D), _row_block(1)],
    out_shape=[
        jax.ShapeDtypeStruct((NP, D), jnp.float32),
        jax.ShapeDtypeStruct((NP, 1), jnp.float32),
    ],
)

_mm2 = pl.pallas_call(
    _mm2_body,
    grid=_GRID,
    in_specs=[_P128_SPEC, _row_block(D), _row_block(1), _BIAS_SPEC, _W_SPEC],
    out_specs=_row_block(D),
    out_shape=jax.ShapeDtypeStruct((NP, D), jnp.float32),
)

_mm3 = pl.pallas_call(
    _mm3_body,
    grid=_GRID,
    in_specs=[_P128_SPEC, _row_block(D), _row_block(1), _BIAS_SPEC, _W_SPEC,
              _BIAS_SPEC],
    out_specs=_row_block(D),
    out_shape=jax.ShapeDtypeStruct((NP, D), jnp.float32),
)


def kernel(x, edge_index, W1, b1, W2, b2, Wm, bm):
    ei = edge_index.astype(jnp.int32)
    src3 = ei[0].reshape(NW, CH, K)
    dst3 = ei[1].reshape(NW, CH, K)
    ones128 = jnp.ones((K, D), jnp.float32)
    z128 = jnp.zeros((RS, D), jnp.float32)

    xp = jnp.pad(x, ((0, NP - N), (0, 0)))
    cnt = _sc_count(dst3, ones128, z128)
    h1p, dinv = _mm1(cnt, xp, W1)
    p1 = _sc_agg(h1p, src3, dst3, z128)
    h2p = _mm2(p1, h1p, dinv, b1.reshape(1, D), W2)
    p2 = _sc_agg(h2p, src3, dst3, z128)
    out = _mm3(p2, h2p, dinv, b2.reshape(1, D), Wm, bm.reshape(1, D))
    return out[:N]
